# Initial kernel scaffold; baseline (speedup 1.0000x reference)
import jax
import jax.numpy as jnp
from jax.experimental import pallas as pl


def kernel(x, edge_index, batch, W1, b1, W2, b2, fc1_W, fc1_b, fc2_W, fc2_b):
    raise NotImplementedError("write your pallas kernel here")


# speedup vs baseline: 21.6553x; 21.6553x over previous
import functools

import jax
import jax.numpy as jnp
from jax import lax
from jax.experimental import pallas as pl
from jax.experimental.pallas import tpu as pltpu
from jax.experimental.pallas import tpu_sc as plsc

N = 10000
E = 320000
G = 512
D_IN = 128
H = 32

NC = 2
NS = 16
NW = NC * NS
L = 16

KE = 79
EPAD = NW * KE * 128
KN = 3
NPAD = NW * KN * 128

NP = 10240
NP_T = NP // NS
GP = 640
GP_T = GP // NS


def _fill_zeros(ref, nrows):
    z16 = jnp.zeros((L,), jnp.float32)
    w = ref.shape[1]

    def body(i, _):
        for j0 in range(0, w, L):
            ref[i, j0:j0 + L] = z16
        return 0

    lax.fori_loop(0, nrows, body, 0)


def _fill_ones(ref, nrows):
    o16 = jnp.ones((L,), jnp.float32)

    def body(i, _):
        ref[i, 0:L] = o16
        return 0

    lax.fori_loop(0, nrows, body, 0)


@functools.cache
def _sc_kernels():
    mesh = plsc.VectorSubcoreMesh(core_axis_name="c", subcore_axis_name="s")

    @functools.partial(
        pl.kernel,
        out_type=[
            jax.ShapeDtypeStruct((NC, NP, L), jnp.float32),
            jax.ShapeDtypeStruct((NC, GP, L), jnp.float32),
        ],
        mesh=mesh,
        compiler_params=pltpu.CompilerParams(use_tc_tiling_on_sc=False),
        scratch_types=[
            pltpu.VMEM((KE, 128), jnp.int32),
            pltpu.VMEM((KN, 128), jnp.int32),
            pltpu.VMEM((128, L), jnp.float32),
            pltpu.VMEM((128, L), jnp.float32),
            pltpu.VMEM_SHARED((NP, L), jnp.float32),
            pltpu.VMEM_SHARED((GP, L), jnp.float32),
        ],
    )
    def _deg_cnt(dst_hbm, bat_hbm, deg_out, cnt_out,
                 dstv, batv, ones_v, zero_v, deg_sh, cnt_sh):
        c = lax.axis_index("c")
        s = lax.axis_index("s")
        wid = s * NC + c
        pltpu.sync_copy(dst_hbm.at[wid], dstv)
        pltpu.sync_copy(bat_hbm.at[wid], batv)
        _fill_ones(ones_v, 128)
        _fill_zeros(zero_v, 128)
        nbase = s * NP_T
        for j in range(NP_T // 128):
            pltpu.sync_copy(zero_v, deg_sh.at[pl.ds(nbase + j * 128, 128)])
        gbase = s * GP_T
        pltpu.sync_copy(zero_v.at[pl.ds(0, GP_T)],
                        cnt_sh.at[pl.ds(gbase, GP_T)])
        plsc.subcore_barrier()

        def body(j, _):
            pltpu.sync_copy(ones_v, deg_sh.at[dstv.at[j]], add=True)
            return 0

        lax.fori_loop(0, KE, body, 0)
        for j in range(KN):
            pltpu.sync_copy(ones_v, cnt_sh.at[batv.at[j]], add=True)
        plsc.subcore_barrier()
        pltpu.sync_copy(deg_sh.at[pl.ds(nbase, NP_T)],
                        deg_out.at[c, pl.ds(nbase, NP_T)])
        pltpu.sync_copy(cnt_sh.at[pl.ds(gbase, GP_T)],
                        cnt_out.at[c, pl.ds(gbase, GP_T)])

    @functools.partial(
        pl.kernel,
        out_type=jax.ShapeDtypeStruct((NC, NP, H), jnp.float32),
        mesh=mesh,
        compiler_params=pltpu.CompilerParams(use_tc_tiling_on_sc=False),
        scratch_types=[
            pltpu.VMEM((KE, 128), jnp.int32),
            pltpu.VMEM((KE, 128), jnp.int32),
            pltpu.VMEM((128, H), jnp.float32),
            pltpu.VMEM((128, H), jnp.float32),
            pltpu.VMEM_SHARED((NP, H), jnp.float32),
            pltpu.SemaphoreType.DMA,
        ],
    )
    def _agg(tab_hbm, src_hbm, dst_hbm, out_hbm,
             srcv, dstv, rows_v, zero_v, acc_sh, sem):
        c = lax.axis_index("c")
        s = lax.axis_index("s")
        wid = s * NC + c
        pltpu.sync_copy(src_hbm.at[wid], srcv)
        pltpu.sync_copy(dst_hbm.at[wid], dstv)
        _fill_zeros(zero_v, 128)
        nbase = s * NP_T
        for j in range(NP_T // 128):
            pltpu.sync_copy(zero_v, acc_sh.at[pl.ds(nbase + j * 128, 128)])
        plsc.subcore_barrier()

        def body(j, _):
            pltpu.async_copy(tab_hbm.at[srcv.at[j]], rows_v, sem).wait()
            pltpu.sync_copy(rows_v, acc_sh.at[dstv.at[j]], add=True)
            return 0

        lax.fori_loop(0, KE, body, 0)
        plsc.subcore_barrier()
        pltpu.sync_copy(acc_sh.at[pl.ds(nbase, NP_T)],
                        out_hbm.at[c, pl.ds(nbase, NP_T)])

    @functools.partial(
        pl.kernel,
        out_type=jax.ShapeDtypeStruct((NC, GP, H), jnp.float32),
        mesh=mesh,
        compiler_params=pltpu.CompilerParams(use_tc_tiling_on_sc=False),
        scratch_types=[
            pltpu.VMEM((KN, 128), jnp.int32),
            pltpu.VMEM((128, H), jnp.float32),
            pltpu.VMEM((GP_T, H), jnp.float32),
            pltpu.VMEM_SHARED((GP, H), jnp.float32),
        ],
    )
    def _pool(tab_hbm, bat_hbm, out_hbm, batv, rows_v, zero_v, acc_sh):
        c = lax.axis_index("c")
        s = lax.axis_index("s")
        wid = s * NC + c
        pltpu.sync_copy(bat_hbm.at[wid], batv)
        _fill_zeros(zero_v, GP_T)
        gbase = s * GP_T
        pltpu.sync_copy(zero_v, acc_sh.at[pl.ds(gbase, GP_T)])
        plsc.subcore_barrier()
        for j in range(KN):
            pltpu.sync_copy(
                tab_hbm.at[pl.ds(wid * (KN * 128) + j * 128, 128)], rows_v)
            pltpu.sync_copy(rows_v, acc_sh.at[batv.at[j]], add=True)
        plsc.subcore_barrier()
        pltpu.sync_copy(acc_sh.at[pl.ds(gbase, GP_T)],
                        out_hbm.at[c, pl.ds(gbase, GP_T)])

    return _deg_cnt, _agg, _pool


_RB = 2000


def _tc_h1(x_ref, dp_ref, w_ref, out_ref):
    d = dp_ref[...]
    dinv = lax.rsqrt(d[:, 0] + d[:, 1] + 1.0)
    h = jnp.dot(x_ref[...], w_ref[...], preferred_element_type=jnp.float32)
    out_ref[...] = h * dinv[:, None]


def _tc_mid(p0_ref, p1_ref, hp_ref, dp_ref, w_ref, b_ref, out_ref):
    d = dp_ref[...]
    dinv = lax.rsqrt(d[:, 0] + d[:, 1] + 1.0)
    agg = (p0_ref[...] + p1_ref[...] + hp_ref[...]) * dinv[:, None]
    a1 = jnp.maximum(agg + b_ref[...], 0.0)
    out_ref[...] = jnp.dot(a1, w_ref[...],
                           preferred_element_type=jnp.float32) * dinv[:, None]


def _tc_h2(q0_ref, q1_ref, hp_ref, dp_ref, b_ref, out_ref):
    d = dp_ref[...]
    dinv = lax.rsqrt(d[:, 0] + d[:, 1] + 1.0)
    out_ref[...] = (q0_ref[...] + q1_ref[...] + hp_ref[...]) * dinv[:, None] \
        + b_ref[...]


def _tc_head(s0_ref, s1_ref, cnt_ref, w1_ref, b1_ref, w2_ref, b2_ref, out_ref):
    cnt = jnp.maximum(cnt_ref[0] + cnt_ref[1], 1.0)
    pooled = (s0_ref[...] + s1_ref[...]) / cnt[:, None]
    t = jnp.maximum(
        jnp.dot(pooled, w1_ref[...], preferred_element_type=jnp.float32)
        + b1_ref[...], 0.0)
    out_ref[...] = jnp.dot(t, w2_ref[...],
                           preferred_element_type=jnp.float32) + b2_ref[...]


def _row_spec(width):
    return pl.BlockSpec((_RB, width), lambda i: (i, 0))


def _whole(shape):
    return pl.BlockSpec(shape, lambda *_: tuple(0 for _ in shape))


def kernel(x, edge_index, batch, W1, b1, W2, b2, fc1_W, fc1_b, fc2_W, fc2_b):
    _deg_cnt, _agg, _pool = _sc_kernels()

    src = edge_index[0].astype(jnp.int32)
    dst = edge_index[1].astype(jnp.int32)
    bat = batch.astype(jnp.int32)

    src3 = jnp.concatenate(
        [src, jnp.zeros((EPAD - E,), jnp.int32)]).reshape(NW, KE, 128)
    dst3 = jnp.concatenate(
        [dst, jnp.full((EPAD - E,), N, jnp.int32)]).reshape(NW, KE, 128)
    bat3 = jnp.concatenate(
        [bat, jnp.full((NPAD - N,), G, jnp.int32)]).reshape(NW, KN, 128)

    deg_p, cnt_p = _deg_cnt(dst3, bat3)
    dp = jnp.transpose(deg_p[:, :N, 0])
    cp = cnt_p[:, :G, 0]

    grid = (N // _RB,)
    h1p = pl.pallas_call(
        _tc_h1,
        grid=grid,
        in_specs=[_row_spec(D_IN), _row_spec(2), _whole((D_IN, H))],
        out_specs=_row_spec(H),
        out_shape=jax.ShapeDtypeStruct((N, H), jnp.float32),
    )(x, dp, W1)

    p = _agg(h1p, src3, dst3)
    h2p = pl.pallas_call(
        _tc_mid,
        grid=grid,
        in_specs=[_row_spec(H), _row_spec(H), _row_spec(H), _row_spec(2),
                  _whole((H, H)), _whole((1, H))],
        out_specs=_row_spec(H),
        out_shape=jax.ShapeDtypeStruct((N, H), jnp.float32),
    )(p[0, :N], p[1, :N], h1p, dp, W2, b1.reshape(1, H))

    q = _agg(h2p, src3, dst3)
    h2 = pl.pallas_call(
        _tc_h2,
        grid=grid,
        in_specs=[_row_spec(H), _row_spec(H), _row_spec(H), _row_spec(2),
                  _whole((1, H))],
        out_specs=_row_spec(H),
        out_shape=jax.ShapeDtypeStruct((N, H), jnp.float32),
    )(q[0, :N], q[1, :N], h2p, dp, b2.reshape(1, H))

    h2pad = jnp.concatenate(
        [h2, jnp.zeros((NPAD - N, H), jnp.float32)])
    sums = _pool(h2pad, bat3)

    out = pl.pallas_call(
        _tc_head,
        in_specs=[_whole((G, H)), _whole((G, H)), _whole((NC, G)),
                  _whole((H, H)), _whole((1, H)), _whole((H, 1)),
                  _whole((1, 1))],
        out_specs=_whole((G, 1)),
        out_shape=jax.ShapeDtypeStruct((G, 1), jnp.float32),
    )(sums[0, :G], sums[1, :G], cp, fc1_W, fc1_b.reshape(1, H),
      fc2_W, fc2_b.reshape(1, 1))
    return out[:, 0]
